# baseline (speedup 1.0000x reference)
import functools

import jax
import jax.numpy as jnp
from jax import lax
from jax.experimental import pallas as pl
from jax.experimental.pallas import tpu as pltpu
from jax.experimental.pallas import tpu_sc as plsc

N = 10000
E = 320000
DIM = 128
H = 8
DH = DIM // H
HID = 512
G = 16
EPS = 1e-5

NC = 2
NS = 16
NW = NC * NS
EPW = E // NW
CHUNK = 80
NCHUNK = EPW // CHUNK
NPAIR = (NCHUNK - 1) // 2
ACC_W = 136
RPT = 624
RTAIL = N - NS * RPT

EBLK = 3200
NEBLK = E // EBLK


def _ln_stats(xv, oh):
    rowsum = jnp.sum(xv, axis=1, keepdims=True)
    rowsq = jnp.sum(xv * xv, axis=1, keepdims=True)
    s = jnp.sum(oh * rowsum, axis=0, keepdims=True)
    q = jnp.sum(oh * rowsq, axis=0, keepdims=True)
    cnt = jnp.sum(oh, axis=0, keepdims=True)
    norm = jnp.maximum(cnt, 1.0) * DIM
    mean = s / norm
    var = q / norm - mean * mean
    rstd = lax.rsqrt(var + EPS)
    mean_n = jnp.sum(oh * mean, axis=1, keepdims=True)
    rstd_n = jnp.sum(oh * rstd, axis=1, keepdims=True)
    return (xv - mean_n) * rstd_n


def _pre_body(x_ref, nidx_ref, w_ref, b_ref, qkvw_ref, qkvb_ref,
              q_ref, kv_ref):
    xv = x_ref[...]
    oh = (nidx_ref[...] == lax.broadcasted_iota(jnp.int32, (1, G), 1))
    oh = oh.astype(jnp.float32)
    h = _ln_stats(xv, oh) * w_ref[...] + b_ref[...]
    qkv = jnp.dot(h, qkvw_ref[...],
                  preferred_element_type=jnp.float32) + qkvb_ref[...]
    q_ref[...] = (qkv[:, :DIM] * (DH ** -0.5)).astype(jnp.bfloat16)
    kv_ref[...] = qkv[:, DIM:].astype(jnp.bfloat16)


def _tc_pre(x, nidx, w, b, qkvw, qkvb):
    return pl.pallas_call(
        _pre_body,
        out_shape=[
            jax.ShapeDtypeStruct((N, DIM), jnp.bfloat16),
            jax.ShapeDtypeStruct((N, 2 * DIM), jnp.bfloat16),
        ],
        compiler_params=pltpu.CompilerParams(
            vmem_limit_bytes=100 * 1024 * 1024),
    )(x, nidx, w, b, qkvw, qkvb)


def _sc_gather_body(q_hbm, kv_hbm, dst_hbm, src_hbm, qe_hbm, kve_hbm,
                    dsta, srca, qrows0, qrows1, kvrows0, kvrows1,
                    semq0, semq1, semk0, semk1):
    cid = lax.axis_index("c")
    sid = lax.axis_index("s")
    wid = sid * NC + cid
    ebase = wid * EPW

    pltpu.sync_copy(dst_hbm.at[pl.ds(ebase, EPW)], dsta)
    pltpu.sync_copy(src_hbm.at[pl.ds(ebase, EPW)], srca)

    def islice(i):
        return pl.ds(pl.multiple_of(i * CHUNK, 8), CHUNK)

    def fire(i, qr, kvr, sq, sk):
        s = islice(i)
        pltpu.async_copy(q_hbm.at[dsta.at[s]], qr, sq)
        pltpu.async_copy(kv_hbm.at[srca.at[s]], kvr, sk)

    def drain(i, qr, kvr, sq, sk):
        s = islice(i)
        pltpu.make_async_copy(q_hbm.at[dsta.at[s]], qr, sq).wait()
        pltpu.make_async_copy(kv_hbm.at[srca.at[s]], kvr, sk).wait()
        eb = ebase + i * CHUNK
        pltpu.sync_copy(qr, qe_hbm.at[pl.ds(eb, CHUNK)])
        pltpu.sync_copy(kvr, kve_hbm.at[pl.ds(eb, CHUNK)])

    fire(0, qrows0, kvrows0, semq0, semk0)

    def body(j, carry):
        i0 = 2 * j
        fire(i0 + 1, qrows1, kvrows1, semq1, semk1)
        drain(i0, qrows0, kvrows0, semq0, semk0)
        fire(i0 + 2, qrows0, kvrows0, semq0, semk0)
        drain(i0 + 1, qrows1, kvrows1, semq1, semk1)
        return carry

    lax.fori_loop(0, NPAIR, body, 0)
    drain(NCHUNK - 1, qrows0, kvrows0, semq0, semk0)


def _sc_gather(q, kv, dst, src):
    mesh = plsc.VectorSubcoreMesh(core_axis_name="c", subcore_axis_name="s")
    f = functools.partial(
        pl.kernel,
        mesh=mesh,
        compiler_params=pltpu.CompilerParams(
            use_tc_tiling_on_sc=False, needs_layout_passes=False),
        out_type=[
            jax.ShapeDtypeStruct((E, DIM), jnp.bfloat16),
            jax.ShapeDtypeStruct((E, 2 * DIM), jnp.bfloat16),
        ],
        scratch_types=[
            pltpu.VMEM((EPW,), jnp.int32),
            pltpu.VMEM((EPW,), jnp.int32),
            pltpu.VMEM((CHUNK, DIM), jnp.bfloat16),
            pltpu.VMEM((CHUNK, DIM), jnp.bfloat16),
            pltpu.VMEM((CHUNK, 2 * DIM), jnp.bfloat16),
            pltpu.VMEM((CHUNK, 2 * DIM), jnp.bfloat16),
            pltpu.SemaphoreType.DMA,
            pltpu.SemaphoreType.DMA,
            pltpu.SemaphoreType.DMA,
            pltpu.SemaphoreType.DMA,
        ],
    )(_sc_gather_body)
    return f(q, kv, dst, src)


def _mid_body(qe_ref, kve_ref, o_ref):
    qe = qe_ref[...].astype(jnp.float32)
    ke = kve_ref[:, :DIM].astype(jnp.float32)
    ve = kve_ref[:, DIM:].astype(jnp.float32)
    em = (lax.broadcasted_iota(jnp.int32, (DIM, H), 1)
          == lax.broadcasted_iota(jnp.int32, (DIM, H), 0) // DH)
    em = em.astype(jnp.float32)
    s8 = jnp.dot(qe * ke, em, preferred_element_type=jnp.float32)
    e8 = jnp.exp(s8)
    evb = jnp.dot(e8, em.T, preferred_element_type=jnp.float32)
    o_ref[:, :DIM] = ve * evb
    o_ref[:, DIM:] = e8


def _tc_mid(qe, kve):
    return pl.pallas_call(
        _mid_body,
        grid=(NEBLK,),
        in_specs=[
            pl.BlockSpec((EBLK, DIM), lambda i: (i, 0)),
            pl.BlockSpec((EBLK, 2 * DIM), lambda i: (i, 0)),
        ],
        out_specs=pl.BlockSpec((EBLK, ACC_W), lambda i: (i, 0)),
        out_shape=jax.ShapeDtypeStruct((E, ACC_W), jnp.float32),
        compiler_params=pltpu.CompilerParams(
            dimension_semantics=("arbitrary",),
            vmem_limit_bytes=100 * 1024 * 1024),
    )(qe, kve)


def _sc_scatter_body(contrib_hbm, dst_hbm, zeros_hbm, out_hbm,
                     dsta, crows0, crows1, sem0, sem1, acc):
    cid = lax.axis_index("c")
    sid = lax.axis_index("s")
    wid = sid * NC + cid
    ebase = wid * EPW

    pltpu.sync_copy(zeros_hbm.at[pl.ds(sid * RPT, RPT)],
                    acc.at[pl.ds(sid * RPT, RPT)])

    @pl.when(sid == 0)
    def _zero_tail():
        pltpu.sync_copy(zeros_hbm.at[pl.ds(NS * RPT, RTAIL)],
                        acc.at[pl.ds(NS * RPT, RTAIL)])

    pltpu.sync_copy(dst_hbm.at[pl.ds(ebase, EPW)], dsta)
    plsc.subcore_barrier()

    def islice(i):
        return pl.ds(pl.multiple_of(i * CHUNK, 8), CHUNK)

    def fire(i, cr, sem):
        eb = ebase + i * CHUNK
        pltpu.async_copy(contrib_hbm.at[pl.ds(eb, CHUNK)], cr, sem)

    def drain(i, cr, sem):
        eb = ebase + i * CHUNK
        pltpu.make_async_copy(contrib_hbm.at[pl.ds(eb, CHUNK)],
                              cr, sem).wait()
        pltpu.sync_copy(cr, acc.at[dsta.at[islice(i)]], add=True)

    fire(0, crows0, sem0)

    def body(j, carry):
        i0 = 2 * j
        fire(i0 + 1, crows1, sem1)
        drain(i0, crows0, sem0)
        fire(i0 + 2, crows0, sem0)
        drain(i0 + 1, crows1, sem1)
        return carry

    lax.fori_loop(0, NPAIR, body, 0)
    drain(NCHUNK - 1, crows0, sem0)

    plsc.subcore_barrier()

    pltpu.sync_copy(acc.at[pl.ds(sid * RPT, RPT)],
                    out_hbm.at[cid, pl.ds(sid * RPT, RPT)])

    @pl.when(sid == 0)
    def _write_tail():
        pltpu.sync_copy(acc.at[pl.ds(NS * RPT, RTAIL)],
                        out_hbm.at[cid, pl.ds(NS * RPT, RTAIL)])


def _sc_scatter(contrib, dst, zeros_init):
    mesh = plsc.VectorSubcoreMesh(core_axis_name="c", subcore_axis_name="s")
    f = functools.partial(
        pl.kernel,
        mesh=mesh,
        compiler_params=pltpu.CompilerParams(
            use_tc_tiling_on_sc=False, needs_layout_passes=False),
        out_type=jax.ShapeDtypeStruct((NC, N, ACC_W), jnp.float32),
        scratch_types=[
            pltpu.VMEM((EPW,), jnp.int32),
            pltpu.VMEM((CHUNK, ACC_W), jnp.float32),
            pltpu.VMEM((CHUNK, ACC_W), jnp.float32),
            pltpu.SemaphoreType.DMA,
            pltpu.SemaphoreType.DMA,
            pltpu.VMEM_SHARED((N, ACC_W), jnp.float32),
        ],
    )(_sc_scatter_body)
    return f(contrib, dst, zeros_init)


def _post_body(x_ref, p0_ref, p1_ref, nidx_ref, pw_ref, pb_ref,
               nw_ref, nb_ref, w1_ref, b1_ref, w2_ref, b2_ref, o_ref):
    num = p0_ref[:, :DIM] + p1_ref[:, :DIM]
    den = p0_ref[:, DIM:DIM + H] + p1_ref[:, DIM:DIM + H]
    em = (lax.broadcasted_iota(jnp.int32, (H, DIM), 1) // DH
          == lax.broadcasted_iota(jnp.int32, (H, DIM), 0))
    den_b = jnp.dot(den, em.astype(jnp.float32),
                    preferred_element_type=jnp.float32)
    attn = jnp.where(den_b > 0.0, num / den_b, 0.0)
    sa = jnp.dot(attn, pw_ref[...],
                 preferred_element_type=jnp.float32) + pb_ref[...]
    x1 = x_ref[...] + sa

    oh = (nidx_ref[...] == lax.broadcasted_iota(jnp.int32, (1, G), 1))
    oh = oh.astype(jnp.float32)
    h2 = _ln_stats(x1, oh) * nw_ref[...] + nb_ref[...]

    g1 = jnp.dot(h2, w1_ref[...],
                 preferred_element_type=jnp.float32) + b1_ref[...]
    ge = 0.5 * g1 * (1.0 + lax.erf(g1 * (2.0 ** -0.5)))
    o_ref[...] = x_ref[...] + jnp.dot(
        ge, w2_ref[...], preferred_element_type=jnp.float32) + b2_ref[...]


def _tc_post(x, p0, p1, nidx, pw, pb, nw, nb, w1, b1, w2, b2):
    return pl.pallas_call(
        _post_body,
        out_shape=jax.ShapeDtypeStruct((N, DIM), jnp.float32),
        compiler_params=pltpu.CompilerParams(
            vmem_limit_bytes=100 * 1024 * 1024),
    )(x, p0, p1, nidx, pw, pb, nw, nb, w1, b1, w2, b2)


def kernel(x, edge_index, norm_index, sa_norm_w, sa_norm_b, qkv_w, qkv_b,
           proj_w, proj_b, ffn_norm_w, ffn_norm_b, ffn_w1, ffn_b1,
           ffn_w2, ffn_b2):
    nidx = norm_index.reshape(N, 1)
    q, kv = _tc_pre(x, nidx, sa_norm_w.reshape(1, DIM),
                    sa_norm_b.reshape(1, DIM), qkv_w,
                    qkv_b.reshape(1, 3 * DIM))
    src = edge_index[0]
    dst = edge_index[1]
    qe, kve = _sc_gather(q, kv, dst, src)
    contrib = _tc_mid(qe, kve)
    zeros_init = jnp.zeros((N, ACC_W), jnp.float32)
    parts = _sc_scatter(contrib, dst, zeros_init)
    out = _tc_post(x, parts[0], parts[1], nidx, proj_w,
                   proj_b.reshape(1, DIM), ffn_norm_w.reshape(1, DIM),
                   ffn_norm_b.reshape(1, DIM), ffn_w1,
                   ffn_b1.reshape(1, HID), ffn_w2, ffn_b2.reshape(1, DIM))
    return out


# speedup vs baseline: 1.1840x; 1.1840x over previous
import functools

import jax
import jax.numpy as jnp
from jax import lax
from jax.experimental import pallas as pl
from jax.experimental.pallas import tpu as pltpu
from jax.experimental.pallas import tpu_sc as plsc

N = 10000
E = 320000
DIM = 128
H = 8
DH = DIM // H
HID = 512
G = 16
EPS = 1e-5

NC = 2
NS = 16
NW = NC * NS
S = 5
ESL = E // S
EPW = ESL // NW
CHUNK = 80
NCHUNK = EPW // CHUNK
NPAIR = (NCHUNK - 1) // 2
ACC_W = 136
RPT = 624
RTAIL = N - NS * RPT

EBLK = 3200
NEBLK = ESL // EBLK


def _ln_stats(xv, oh):
    rowsum = jnp.sum(xv, axis=1, keepdims=True)
    rowsq = jnp.sum(xv * xv, axis=1, keepdims=True)
    s = jnp.sum(oh * rowsum, axis=0, keepdims=True)
    q = jnp.sum(oh * rowsq, axis=0, keepdims=True)
    cnt = jnp.sum(oh, axis=0, keepdims=True)
    norm = jnp.maximum(cnt, 1.0) * DIM
    mean = s / norm
    var = q / norm - mean * mean
    rstd = lax.rsqrt(var + EPS)
    mean_n = jnp.sum(oh * mean, axis=1, keepdims=True)
    rstd_n = jnp.sum(oh * rstd, axis=1, keepdims=True)
    return (xv - mean_n) * rstd_n


def _pre_body(x_ref, nidx_ref, w_ref, b_ref, qkvw_ref, qkvb_ref,
              q_ref, kv_ref):
    xv = x_ref[...]
    oh = (nidx_ref[...] == lax.broadcasted_iota(jnp.int32, (1, G), 1))
    oh = oh.astype(jnp.float32)
    h = _ln_stats(xv, oh) * w_ref[...] + b_ref[...]
    qkv = jnp.dot(h, qkvw_ref[...],
                  preferred_element_type=jnp.float32) + qkvb_ref[...]
    q_ref[...] = qkv[:, :DIM] * (DH ** -0.5)
    kv_ref[...] = qkv[:, DIM:]


def _tc_pre(x, nidx, w, b, qkvw, qkvb):
    return pl.pallas_call(
        _pre_body,
        out_shape=[
            jax.ShapeDtypeStruct((N, DIM), jnp.float32),
            jax.ShapeDtypeStruct((N, 2 * DIM), jnp.float32),
        ],
        compiler_params=pltpu.CompilerParams(
            vmem_limit_bytes=100 * 1024 * 1024),
    )(x, nidx, w, b, qkvw, qkvb)


def _sc_gather_body(q_hbm, kv_hbm, dst_hbm, src_hbm, qe_hbm, kve_hbm,
                    dsta, srca, qrows0, qrows1, kvrows0, kvrows1,
                    semq0, semq1, semk0, semk1):
    cid = lax.axis_index("c")
    sid = lax.axis_index("s")
    wid = sid * NC + cid
    ebase = wid * EPW

    pltpu.sync_copy(dst_hbm.at[pl.ds(ebase, EPW)], dsta)
    pltpu.sync_copy(src_hbm.at[pl.ds(ebase, EPW)], srca)

    def islice(i):
        return pl.ds(pl.multiple_of(i * CHUNK, 8), CHUNK)

    def fire(i, qr, kvr, sq, sk):
        s = islice(i)
        pltpu.async_copy(q_hbm.at[dsta.at[s]], qr, sq)
        pltpu.async_copy(kv_hbm.at[srca.at[s]], kvr, sk)

    def drain(i, qr, kvr, sq, sk):
        s = islice(i)
        pltpu.make_async_copy(q_hbm.at[dsta.at[s]], qr, sq).wait()
        pltpu.make_async_copy(kv_hbm.at[srca.at[s]], kvr, sk).wait()
        eb = ebase + i * CHUNK
        pltpu.sync_copy(qr, qe_hbm.at[pl.ds(eb, CHUNK)])
        pltpu.sync_copy(kvr, kve_hbm.at[pl.ds(eb, CHUNK)])

    fire(0, qrows0, kvrows0, semq0, semk0)

    def body(j, carry):
        i0 = 2 * j
        fire(i0 + 1, qrows1, kvrows1, semq1, semk1)
        drain(i0, qrows0, kvrows0, semq0, semk0)
        fire(i0 + 2, qrows0, kvrows0, semq0, semk0)
        drain(i0 + 1, qrows1, kvrows1, semq1, semk1)
        return carry

    lax.fori_loop(0, NPAIR, body, 0)
    drain(NCHUNK - 1, qrows0, kvrows0, semq0, semk0)


def _sc_gather(q, kv, dst, src):
    mesh = plsc.VectorSubcoreMesh(core_axis_name="c", subcore_axis_name="s")
    f = functools.partial(
        pl.kernel,
        mesh=mesh,
        compiler_params=pltpu.CompilerParams(
            use_tc_tiling_on_sc=False, needs_layout_passes=False),
        out_type=[
            jax.ShapeDtypeStruct((ESL, DIM), jnp.float32),
            jax.ShapeDtypeStruct((ESL, 2 * DIM), jnp.float32),
        ],
        scratch_types=[
            pltpu.VMEM((EPW,), jnp.int32),
            pltpu.VMEM((EPW,), jnp.int32),
            pltpu.VMEM((CHUNK, DIM), jnp.float32),
            pltpu.VMEM((CHUNK, DIM), jnp.float32),
            pltpu.VMEM((CHUNK, 2 * DIM), jnp.float32),
            pltpu.VMEM((CHUNK, 2 * DIM), jnp.float32),
            pltpu.SemaphoreType.DMA,
            pltpu.SemaphoreType.DMA,
            pltpu.SemaphoreType.DMA,
            pltpu.SemaphoreType.DMA,
        ],
    )(_sc_gather_body)
    return f(q, kv, dst, src)


def _mid_body(qe_ref, kve_ref, o_ref):
    qe = qe_ref[...]
    ke = kve_ref[:, :DIM]
    ve = kve_ref[:, DIM:]
    em = (lax.broadcasted_iota(jnp.int32, (DIM, H), 1)
          == lax.broadcasted_iota(jnp.int32, (DIM, H), 0) // DH)
    em = em.astype(jnp.float32)
    s8 = jnp.dot(qe * ke, em, preferred_element_type=jnp.float32)
    e8 = jnp.exp(s8)
    evb = jnp.dot(e8, em.T, preferred_element_type=jnp.float32)
    o_ref[:, :DIM] = ve * evb
    o_ref[:, DIM:] = e8


def _tc_mid(qe, kve):
    return pl.pallas_call(
        _mid_body,
        grid=(NEBLK,),
        in_specs=[
            pl.BlockSpec((EBLK, DIM), lambda i: (i, 0)),
            pl.BlockSpec((EBLK, 2 * DIM), lambda i: (i, 0)),
        ],
        out_specs=pl.BlockSpec((EBLK, ACC_W), lambda i: (i, 0)),
        out_shape=jax.ShapeDtypeStruct((ESL, ACC_W), jnp.float32),
        compiler_params=pltpu.CompilerParams(
            dimension_semantics=("arbitrary",),
            vmem_limit_bytes=100 * 1024 * 1024),
    )(qe, kve)


def _sc_scatter_body(c0, c1, c2, c3, c4, d0, d1, d2, d3, d4,
                     zeros_hbm, out_hbm,
                     dsta, crows0, crows1, sem0, sem1, acc):
    cid = lax.axis_index("c")
    sid = lax.axis_index("s")
    wid = sid * NC + cid
    ebase = wid * EPW

    pltpu.sync_copy(zeros_hbm.at[pl.ds(sid * RPT, RPT)],
                    acc.at[pl.ds(sid * RPT, RPT)])

    @pl.when(sid == 0)
    def _zero_tail():
        pltpu.sync_copy(zeros_hbm.at[pl.ds(NS * RPT, RTAIL)],
                        acc.at[pl.ds(NS * RPT, RTAIL)])

    plsc.subcore_barrier()

    def islice(i):
        return pl.ds(pl.multiple_of(i * CHUNK, 8), CHUNK)

    for contrib_hbm, dst_hbm in ((c0, d0), (c1, d1), (c2, d2),
                                 (c3, d3), (c4, d4)):
        pltpu.sync_copy(dst_hbm.at[pl.ds(ebase, EPW)], dsta)

        def fire(i, cr, sem):
            eb = ebase + i * CHUNK
            pltpu.async_copy(contrib_hbm.at[pl.ds(eb, CHUNK)], cr, sem)

        def drain(i, cr, sem):
            eb = ebase + i * CHUNK
            pltpu.make_async_copy(contrib_hbm.at[pl.ds(eb, CHUNK)],
                                  cr, sem).wait()
            pltpu.sync_copy(cr, acc.at[dsta.at[islice(i)]], add=True)

        fire(0, crows0, sem0)

        def body(j, carry):
            i0 = 2 * j
            fire(i0 + 1, crows1, sem1)
            drain(i0, crows0, sem0)
            fire(i0 + 2, crows0, sem0)
            drain(i0 + 1, crows1, sem1)
            return carry

        lax.fori_loop(0, NPAIR, body, 0)
        drain(NCHUNK - 1, crows0, sem0)

    plsc.subcore_barrier()

    pltpu.sync_copy(acc.at[pl.ds(sid * RPT, RPT)],
                    out_hbm.at[cid, pl.ds(sid * RPT, RPT)])

    @pl.when(sid == 0)
    def _write_tail():
        pltpu.sync_copy(acc.at[pl.ds(NS * RPT, RTAIL)],
                        out_hbm.at[cid, pl.ds(NS * RPT, RTAIL)])


def _sc_scatter(contribs, dsts, zeros_init):
    mesh = plsc.VectorSubcoreMesh(core_axis_name="c", subcore_axis_name="s")
    f = functools.partial(
        pl.kernel,
        mesh=mesh,
        compiler_params=pltpu.CompilerParams(
            use_tc_tiling_on_sc=False, needs_layout_passes=False),
        out_type=jax.ShapeDtypeStruct((NC, N, ACC_W), jnp.float32),
        scratch_types=[
            pltpu.VMEM((EPW,), jnp.int32),
            pltpu.VMEM((CHUNK, ACC_W), jnp.float32),
            pltpu.VMEM((CHUNK, ACC_W), jnp.float32),
            pltpu.SemaphoreType.DMA,
            pltpu.SemaphoreType.DMA,
            pltpu.VMEM_SHARED((N, ACC_W), jnp.float32),
        ],
    )(_sc_scatter_body)
    return f(*contribs, *dsts, zeros_init)


def _post_body(x_ref, p0_ref, p1_ref, nidx_ref, pw_ref, pb_ref,
               nw_ref, nb_ref, w1_ref, b1_ref, w2_ref, b2_ref, o_ref):
    num = p0_ref[:, :DIM] + p1_ref[:, :DIM]
    den = p0_ref[:, DIM:DIM + H] + p1_ref[:, DIM:DIM + H]
    em = (lax.broadcasted_iota(jnp.int32, (H, DIM), 1) // DH
          == lax.broadcasted_iota(jnp.int32, (H, DIM), 0))
    den_b = jnp.dot(den, em.astype(jnp.float32),
                    preferred_element_type=jnp.float32)
    attn = jnp.where(den_b > 0.0, num / den_b, 0.0)
    sa = jnp.dot(attn, pw_ref[...],
                 preferred_element_type=jnp.float32) + pb_ref[...]
    x1 = x_ref[...] + sa

    oh = (nidx_ref[...] == lax.broadcasted_iota(jnp.int32, (1, G), 1))
    oh = oh.astype(jnp.float32)
    h2 = _ln_stats(x1, oh) * nw_ref[...] + nb_ref[...]

    g1 = jnp.dot(h2, w1_ref[...],
                 preferred_element_type=jnp.float32) + b1_ref[...]
    ge = 0.5 * g1 * (1.0 + lax.erf(g1 * (2.0 ** -0.5)))
    o_ref[...] = x_ref[...] + jnp.dot(
        ge, w2_ref[...], preferred_element_type=jnp.float32) + b2_ref[...]


def _tc_post(x, p0, p1, nidx, pw, pb, nw, nb, w1, b1, w2, b2):
    return pl.pallas_call(
        _post_body,
        out_shape=jax.ShapeDtypeStruct((N, DIM), jnp.float32),
        compiler_params=pltpu.CompilerParams(
            vmem_limit_bytes=100 * 1024 * 1024),
    )(x, p0, p1, nidx, pw, pb, nw, nb, w1, b1, w2, b2)


def kernel(x, edge_index, norm_index, sa_norm_w, sa_norm_b, qkv_w, qkv_b,
           proj_w, proj_b, ffn_norm_w, ffn_norm_b, ffn_w1, ffn_b1,
           ffn_w2, ffn_b2):
    nidx = norm_index.reshape(N, 1)
    q, kv = _tc_pre(x, nidx, sa_norm_w.reshape(1, DIM),
                    sa_norm_b.reshape(1, DIM), qkv_w,
                    qkv_b.reshape(1, 3 * DIM))
    src = edge_index[0]
    dst = edge_index[1]
    srcs = [lax.slice(src, (s * ESL,), ((s + 1) * ESL,)) for s in range(S)]
    dsts = [lax.slice(dst, (s * ESL,), ((s + 1) * ESL,)) for s in range(S)]
    contribs = []
    for s in range(S):
        qe, kve = _sc_gather(q, kv, dsts[s], srcs[s])
        contribs.append(_tc_mid(qe, kve))
    zeros_init = jnp.zeros((N, ACC_W), jnp.float32)
    parts = _sc_scatter(contribs, dsts, zeros_init)
    out = _tc_post(x, parts[0], parts[1], nidx, proj_w,
                   proj_b.reshape(1, DIM), ffn_norm_w.reshape(1, DIM),
                   ffn_norm_b.reshape(1, DIM), ffn_w1,
                   ffn_b1.reshape(1, HID), ffn_w2, ffn_b2.reshape(1, DIM))
    return out


# speedup vs baseline: 1.9059x; 1.6097x over previous
import functools

import jax
import jax.numpy as jnp
from jax import lax
from jax.experimental import pallas as pl
from jax.experimental.pallas import tpu as pltpu
from jax.experimental.pallas import tpu_sc as plsc

N = 10000
E = 320000
DIM = 128
H = 8
DH = DIM // H
HID = 512
G = 16
EPS = 1e-5

NC = 2
NS = 16
NW = NC * NS
S = 5
ESL = E // S
EPW = ESL // NW
CHUNK = 80
NCHUNK = EPW // CHUNK
NPAIR = (NCHUNK - 1) // 2
RPT = 624
RTAIL = N - NS * RPT

EBLK = 3200
NEBLK = ESL // EBLK


def _ln_stats(xv, oh):
    rowsum = jnp.sum(xv, axis=1, keepdims=True)
    rowsq = jnp.sum(xv * xv, axis=1, keepdims=True)
    s = jnp.sum(oh * rowsum, axis=0, keepdims=True)
    q = jnp.sum(oh * rowsq, axis=0, keepdims=True)
    cnt = jnp.sum(oh, axis=0, keepdims=True)
    norm = jnp.maximum(cnt, 1.0) * DIM
    mean = s / norm
    var = q / norm - mean * mean
    rstd = lax.rsqrt(var + EPS)
    mean_n = jnp.sum(oh * mean, axis=1, keepdims=True)
    rstd_n = jnp.sum(oh * rstd, axis=1, keepdims=True)
    return (xv - mean_n) * rstd_n


def _pre_body(x_ref, nidx_ref, w_ref, b_ref, qkvw_ref, qkvb_ref,
              q_ref, k_ref, v_ref):
    xv = x_ref[...]
    oh = (nidx_ref[...] == lax.broadcasted_iota(jnp.int32, (1, G), 1))
    oh = oh.astype(jnp.float32)
    h = _ln_stats(xv, oh) * w_ref[...] + b_ref[...]
    qkv = jnp.dot(h, qkvw_ref[...],
                  preferred_element_type=jnp.float32) + qkvb_ref[...]
    q_ref[...] = qkv[:, :DIM] * (DH ** -0.5)
    k_ref[...] = qkv[:, DIM:2 * DIM]
    v_ref[...] = qkv[:, 2 * DIM:]


def _tc_pre(x, nidx, w, b, qkvw, qkvb):
    return pl.pallas_call(
        _pre_body,
        out_shape=[
            jax.ShapeDtypeStruct((N, DIM), jnp.float32),
            jax.ShapeDtypeStruct((N, DIM), jnp.float32),
            jax.ShapeDtypeStruct((N, DIM), jnp.float32),
        ],
        compiler_params=pltpu.CompilerParams(
            vmem_limit_bytes=100 * 1024 * 1024),
    )(x, nidx, w, b, qkvw, qkvb)


def _sc_gather_body(q_hbm, k_hbm, v_hbm, dst_hbm, src_hbm,
                    qe_hbm, ke_hbm, ve_hbm,
                    dsta, srca, qr0, qr1, kr0, kr1, vr0, vr1,
                    sq0, sq1, sk0, sk1, sv0, sv1):
    cid = lax.axis_index("c")
    sid = lax.axis_index("s")
    wid = sid * NC + cid
    ebase = wid * EPW

    pltpu.sync_copy(dst_hbm.at[pl.ds(ebase, EPW)], dsta)
    pltpu.sync_copy(src_hbm.at[pl.ds(ebase, EPW)], srca)

    def islice(i):
        return pl.ds(pl.multiple_of(i * CHUNK, 8), CHUNK)

    def fire(i, qr, kr, vr, sq, sk, sv):
        s = islice(i)
        pltpu.async_copy(q_hbm.at[dsta.at[s]], qr, sq)
        pltpu.async_copy(k_hbm.at[srca.at[s]], kr, sk)
        pltpu.async_copy(v_hbm.at[srca.at[s]], vr, sv)

    def drain(i, qr, kr, vr, sq, sk, sv):
        s = islice(i)
        pltpu.make_async_copy(q_hbm.at[dsta.at[s]], qr, sq).wait()
        pltpu.make_async_copy(k_hbm.at[srca.at[s]], kr, sk).wait()
        pltpu.make_async_copy(v_hbm.at[srca.at[s]], vr, sv).wait()
        eb = ebase + i * CHUNK
        pltpu.sync_copy(qr, qe_hbm.at[pl.ds(eb, CHUNK)])
        pltpu.sync_copy(kr, ke_hbm.at[pl.ds(eb, CHUNK)])
        pltpu.sync_copy(vr, ve_hbm.at[pl.ds(eb, CHUNK)])

    fire(0, qr0, kr0, vr0, sq0, sk0, sv0)

    def body(j, carry):
        i0 = 2 * j
        fire(i0 + 1, qr1, kr1, vr1, sq1, sk1, sv1)
        drain(i0, qr0, kr0, vr0, sq0, sk0, sv0)
        fire(i0 + 2, qr0, kr0, vr0, sq0, sk0, sv0)
        drain(i0 + 1, qr1, kr1, vr1, sq1, sk1, sv1)
        return carry

    lax.fori_loop(0, NPAIR, body, 0)
    drain(NCHUNK - 1, qr0, kr0, vr0, sq0, sk0, sv0)


def _sc_gather(q, k, v, dst, src):
    mesh = plsc.VectorSubcoreMesh(core_axis_name="c", subcore_axis_name="s")
    f = functools.partial(
        pl.kernel,
        mesh=mesh,
        compiler_params=pltpu.CompilerParams(
            use_tc_tiling_on_sc=False, needs_layout_passes=False),
        out_type=[
            jax.ShapeDtypeStruct((ESL, DIM), jnp.float32),
            jax.ShapeDtypeStruct((ESL, DIM), jnp.float32),
            jax.ShapeDtypeStruct((ESL, DIM), jnp.float32),
        ],
        scratch_types=[
            pltpu.VMEM((EPW,), jnp.int32),
            pltpu.VMEM((EPW,), jnp.int32),
            pltpu.VMEM((CHUNK, DIM), jnp.float32),
            pltpu.VMEM((CHUNK, DIM), jnp.float32),
            pltpu.VMEM((CHUNK, DIM), jnp.float32),
            pltpu.VMEM((CHUNK, DIM), jnp.float32),
            pltpu.VMEM((CHUNK, DIM), jnp.float32),
            pltpu.VMEM((CHUNK, DIM), jnp.float32),
            pltpu.SemaphoreType.DMA,
            pltpu.SemaphoreType.DMA,
            pltpu.SemaphoreType.DMA,
            pltpu.SemaphoreType.DMA,
            pltpu.SemaphoreType.DMA,
            pltpu.SemaphoreType.DMA,
        ],
    )(_sc_gather_body)
    return f(q, k, v, dst, src)


def _mid_body(qe_ref, ke_ref, ve_ref, num_ref, den_ref):
    qe = qe_ref[...]
    ke = ke_ref[...]
    ve = ve_ref[...]
    em = (lax.broadcasted_iota(jnp.int32, (DIM, H), 1)
          == lax.broadcasted_iota(jnp.int32, (DIM, H), 0) // DH)
    em = em.astype(jnp.float32)
    s8 = jnp.dot(qe * ke, em, preferred_element_type=jnp.float32)
    e8 = jnp.exp(s8)
    evb = jnp.dot(e8, em.T, preferred_element_type=jnp.float32)
    num_ref[...] = ve * evb
    den_ref[...] = evb


def _tc_mid(qe, ke, ve):
    return pl.pallas_call(
        _mid_body,
        grid=(NEBLK,),
        in_specs=[
            pl.BlockSpec((EBLK, DIM), lambda i: (i, 0)),
            pl.BlockSpec((EBLK, DIM), lambda i: (i, 0)),
            pl.BlockSpec((EBLK, DIM), lambda i: (i, 0)),
        ],
        out_specs=[
            pl.BlockSpec((EBLK, DIM), lambda i: (i, 0)),
            pl.BlockSpec((EBLK, DIM), lambda i: (i, 0)),
        ],
        out_shape=[
            jax.ShapeDtypeStruct((ESL, DIM), jnp.float32),
            jax.ShapeDtypeStruct((ESL, DIM), jnp.float32),
        ],
        compiler_params=pltpu.CompilerParams(
            dimension_semantics=("arbitrary",),
            vmem_limit_bytes=100 * 1024 * 1024),
    )(qe, ke, ve)


def _sc_scatter_body(*refs):
    payloads = refs[0:S]
    dsts = refs[S:2 * S]
    zeros_hbm = refs[2 * S]
    out_hbm = refs[2 * S + 1]
    (dsta, cr0, cr1, sem0, sem1, acc) = refs[2 * S + 2:]

    cid = lax.axis_index("c")
    sid = lax.axis_index("s")
    wid = sid * NC + cid
    ebase = wid * EPW

    pltpu.sync_copy(zeros_hbm.at[pl.ds(sid * RPT, RPT)],
                    acc.at[pl.ds(sid * RPT, RPT)])

    @pl.when(sid == 0)
    def _zero_tail():
        pltpu.sync_copy(zeros_hbm.at[pl.ds(NS * RPT, RTAIL)],
                        acc.at[pl.ds(NS * RPT, RTAIL)])

    plsc.subcore_barrier()

    def islice(i):
        return pl.ds(pl.multiple_of(i * CHUNK, 8), CHUNK)

    for pay_hbm, dst_hbm in zip(payloads, dsts):
        pltpu.sync_copy(dst_hbm.at[pl.ds(ebase, EPW)], dsta)

        def fire(i, cr, sem):
            eb = ebase + i * CHUNK
            pltpu.async_copy(pay_hbm.at[pl.ds(eb, CHUNK)], cr, sem)

        def drain(i, cr, sem):
            eb = ebase + i * CHUNK
            pltpu.make_async_copy(pay_hbm.at[pl.ds(eb, CHUNK)],
                                  cr, sem).wait()
            pltpu.sync_copy(cr, acc.at[dsta.at[islice(i)]], add=True)

        fire(0, cr0, sem0)

        def body(j, carry):
            i0 = 2 * j
            fire(i0 + 1, cr1, sem1)
            drain(i0, cr0, sem0)
            fire(i0 + 2, cr0, sem0)
            drain(i0 + 1, cr1, sem1)
            return carry

        lax.fori_loop(0, NPAIR, body, 0)
        drain(NCHUNK - 1, cr0, sem0)

    plsc.subcore_barrier()

    pltpu.sync_copy(acc.at[pl.ds(sid * RPT, RPT)],
                    out_hbm.at[cid, pl.ds(sid * RPT, RPT)])

    @pl.when(sid == 0)
    def _write_tail():
        pltpu.sync_copy(acc.at[pl.ds(NS * RPT, RTAIL)],
                        out_hbm.at[cid, pl.ds(NS * RPT, RTAIL)])


def _sc_scatter(payloads, dsts, zeros_init):
    mesh = plsc.VectorSubcoreMesh(core_axis_name="c", subcore_axis_name="s")
    f = functools.partial(
        pl.kernel,
        mesh=mesh,
        compiler_params=pltpu.CompilerParams(
            use_tc_tiling_on_sc=False, needs_layout_passes=False),
        out_type=jax.ShapeDtypeStruct((NC, N, DIM), jnp.float32),
        scratch_types=[
            pltpu.VMEM((EPW,), jnp.int32),
            pltpu.VMEM((CHUNK, DIM), jnp.float32),
            pltpu.VMEM((CHUNK, DIM), jnp.float32),
            pltpu.SemaphoreType.DMA,
            pltpu.SemaphoreType.DMA,
            pltpu.VMEM_SHARED((N, DIM), jnp.float32),
        ],
    )(_sc_scatter_body)
    return f(*payloads, *dsts, zeros_init)


def _post_body(x_ref, pn_ref, pd_ref, nidx_ref, pw_ref, pb_ref,
               nw_ref, nb_ref, w1_ref, b1_ref, w2_ref, b2_ref, o_ref):
    num = pn_ref[0] + pn_ref[1]
    den = pd_ref[0] + pd_ref[1]
    attn = jnp.where(den > 0.0, num / den, 0.0)
    sa = jnp.dot(attn, pw_ref[...],
                 preferred_element_type=jnp.float32) + pb_ref[...]
    x1 = x_ref[...] + sa

    oh = (nidx_ref[...] == lax.broadcasted_iota(jnp.int32, (1, G), 1))
    oh = oh.astype(jnp.float32)
    h2 = _ln_stats(x1, oh) * nw_ref[...] + nb_ref[...]

    g1 = jnp.dot(h2, w1_ref[...],
                 preferred_element_type=jnp.float32) + b1_ref[...]
    ge = 0.5 * g1 * (1.0 + lax.erf(g1 * (2.0 ** -0.5)))
    o_ref[...] = x_ref[...] + jnp.dot(
        ge, w2_ref[...], preferred_element_type=jnp.float32) + b2_ref[...]


def _tc_post(x, pn, pd, nidx, pw, pb, nw, nb, w1, b1, w2, b2):
    return pl.pallas_call(
        _post_body,
        out_shape=jax.ShapeDtypeStruct((N, DIM), jnp.float32),
        compiler_params=pltpu.CompilerParams(
            vmem_limit_bytes=100 * 1024 * 1024),
    )(x, pn, pd, nidx, pw, pb, nw, nb, w1, b1, w2, b2)


def kernel(x, edge_index, norm_index, sa_norm_w, sa_norm_b, qkv_w, qkv_b,
           proj_w, proj_b, ffn_norm_w, ffn_norm_b, ffn_w1, ffn_b1,
           ffn_w2, ffn_b2):
    nidx = norm_index.reshape(N, 1)
    q, k, v = _tc_pre(x, nidx, sa_norm_w.reshape(1, DIM),
                      sa_norm_b.reshape(1, DIM), qkv_w,
                      qkv_b.reshape(1, 3 * DIM))
    src = edge_index[0]
    dst = edge_index[1]
    srcs = [lax.slice(src, (s * ESL,), ((s + 1) * ESL,)) for s in range(S)]
    dsts = [lax.slice(dst, (s * ESL,), ((s + 1) * ESL,)) for s in range(S)]
    nums = []
    dens = []
    for s in range(S):
        qe, ke, ve = _sc_gather(q, k, v, dsts[s], srcs[s])
        nm, dn = _tc_mid(qe, ke, ve)
        nums.append(nm)
        dens.append(dn)
    zeros_init = jnp.zeros((N, DIM), jnp.float32)
    pn = _sc_scatter(nums, dsts, zeros_init)
    pd = _sc_scatter(dens, dsts, zeros_init)
    out = _tc_post(x, pn, pd, nidx, proj_w,
                   proj_b.reshape(1, DIM), ffn_norm_w.reshape(1, DIM),
                   ffn_norm_b.reshape(1, DIM), ffn_w1,
                   ffn_b1.reshape(1, HID), ffn_w2, ffn_b2.reshape(1, DIM))
    return out
